# baseline (speedup 1.0000x reference)
import functools
import jax
import jax.numpy as jnp
import numpy as np
from jax import lax
from jax.experimental import pallas as pl
from jax.experimental.pallas import tpu as pltpu
from jax.experimental.pallas import tpu_sc as plsc

_ADD_CH = 3
_CFG = [
    dict(npoint=2048, radius_list=[0.05, 0.1, 0.2], nsample_list=[32, 64, 128],
         mlp_list=[[16, 16, 32], [32, 32, 64], [32, 48, 64]]),
    dict(npoint=512, radius_list=[0.2, 0.4], nsample_list=[64, 128],
         mlp_list=[[64, 64, 128], [64, 96, 128]]),
    dict(npoint=128, radius_list=[0.4, 0.8], nsample_list=[128, 256],
         mlp_list=[[128, 128, 256], [128, 196, 256]]),
]

_NC, _NS = 2, 16
_NW = _NC * _NS


def _round_up(x, m):
    return (x + m - 1) // m * m


def _fps_body(npoint, n_rows, n_cols, xs_ref, ys_ref, zs_ref, idx_ref, dist_ref):
    N = n_rows * n_cols
    rowi = lax.broadcasted_iota(jnp.int32, (n_rows, n_cols), 0)
    coli = lax.broadcasted_iota(jnp.int32, (n_rows, n_cols), 1)
    idxg = rowi * n_cols + coli
    dist_ref[...] = jnp.full((n_rows, n_cols), 1e10, dtype=jnp.float32)

    def step(t, far):
        idx_ref[t] = far
        sel = idxg == far
        cx = jnp.sum(jnp.where(sel, xs_ref[...], 0.0))
        cy = jnp.sum(jnp.where(sel, ys_ref[...], 0.0))
        cz = jnp.sum(jnp.where(sel, zs_ref[...], 0.0))
        dx = xs_ref[...] - cx
        dy = ys_ref[...] - cy
        dz = zs_ref[...] - cz
        d = dx * dx + dy * dy + dz * dz
        nd = jnp.minimum(dist_ref[...], d)
        dist_ref[...] = nd
        mx = jnp.max(nd)
        cand = jnp.where(nd == mx, idxg, N)
        return jnp.min(cand).astype(jnp.int32)

    lax.fori_loop(0, npoint, step, jnp.int32(0))


def _fps(xyz3, npoint, interpret=False):
    B, _, N = xyz3.shape
    n_rows = 8
    n_cols = N // 8
    xs = xyz3[:, 0, :].reshape(B, n_rows, n_cols)
    ys = xyz3[:, 1, :].reshape(B, n_rows, n_cols)
    zs = xyz3[:, 2, :].reshape(B, n_rows, n_cols)
    body = functools.partial(_fps_body, npoint, n_rows, n_cols)
    bs = pl.BlockSpec((1, n_rows, n_cols), lambda b: (b, 0, 0))
    out = pl.pallas_call(
        lambda x, y, z, i, dd: body(x.at[0], y.at[0], z.at[0], i.at[0, 0], dd),
        grid=(B,),
        in_specs=[bs, bs, bs],
        out_specs=pl.BlockSpec((1, 1, npoint), lambda b: (b, 0, 0),
                               memory_space=pltpu.SMEM),
        out_shape=jax.ShapeDtypeStruct((B, 1, npoint), jnp.int32),
        scratch_shapes=[pltpu.VMEM((n_rows, n_cols), jnp.float32)],
        interpret=interpret,
    )(xs, ys, zs)
    return out.reshape(B, npoint)


def _sc_gather(table, idx):
    T, D = table.shape
    F = idx.shape[0]
    assert F % (_NW * 8) == 0
    b_per_w = F // _NW
    chunk = min(128, b_per_w)
    nchunks = b_per_w // chunk
    assert b_per_w % chunk == 0
    depth = min(4, nchunks)
    while depth * chunk * D * 4 > 380 * 1024:
        depth //= 2
    ngroups = nchunks // depth
    assert nchunks % depth == 0
    idx2 = idx.reshape(F // chunk, chunk)
    mesh = plsc.VectorSubcoreMesh(core_axis_name="c", subcore_axis_name="s")

    @functools.partial(
        pl.kernel, mesh=mesh,
        out_type=jax.ShapeDtypeStruct((F, D), jnp.float32),
        scratch_types=[
            pltpu.VMEM((depth, chunk), jnp.int32),
            pltpu.VMEM((depth * chunk, D), jnp.float32),
            pltpu.SemaphoreType.DMA,
        ],
        compiler_params=pltpu.CompilerParams(use_tc_tiling_on_sc=False),
    )
    def k(table_hbm, idx_hbm, out_hbm, idx_v, rows_v, sem):
        wid = lax.axis_index("s") * _NC + lax.axis_index("c")
        gbase = wid * (b_per_w // chunk)

        def grp(g, carry):
            grow = gbase + g * depth
            pltpu.sync_copy(idx_hbm.at[pl.ds(grow, depth)], idx_v)
            cps = [
                pltpu.async_copy(table_hbm.at[idx_v.at[j]],
                                 rows_v.at[pl.ds(j * chunk, chunk)], sem)
                for j in range(depth)
            ]
            for cp in cps:
                cp.wait()
            pltpu.sync_copy(
                rows_v, out_hbm.at[pl.ds(grow * chunk, depth * chunk)])
            return carry

        lax.fori_loop(0, ngroups, grp, jnp.int32(0))

    return k(table, idx2)


def _sc_ballquery(xyz3, cent16, S, radii, Ks):
    B, _, N = xyz3.shape
    CS = (B * S) // _NW
    assert (B * S) % _NW == 0 and S % CS == 0
    r2s = [float(np.float32(r ** 2)) for r in radii]
    mesh = plsc.VectorSubcoreMesh(core_axis_name="c", subcore_axis_name="s")

    scratch = [pltpu.VMEM((4, N), jnp.float32),
               pltpu.VMEM((CS, 16), jnp.float32)]
    for K in Ks:
        scratch.append(pltpu.VMEM((K + 16,), jnp.int32))
    for K in Ks:
        scratch.append(pltpu.VMEM((CS, K), jnp.int32))

    @functools.partial(
        pl.kernel, mesh=mesh,
        out_type=[jax.ShapeDtypeStruct((B * S, K), jnp.int32) for K in Ks],
        scratch_types=scratch,
        compiler_params=pltpu.CompilerParams(needs_layout_passes=False),
    )
    def k(xyz_hbm, cent_hbm, *refs):
        outs = refs[:len(Ks)]
        rq_v = refs[len(Ks)]
        cent_v = refs[len(Ks) + 1]
        bufs = refs[len(Ks) + 2:len(Ks) + 2 + len(Ks)]
        stage = refs[len(Ks) + 2 + len(Ks):]

        def bf16r(x):
            xi = plsc.bitcast(x, jnp.int32)
            r = xi + (jnp.int32(0x7FFF) + ((xi >> 16) & 1))
            return plsc.bitcast(r & jnp.int32(-65536), jnp.float32)

        wid = lax.axis_index("s") * _NC + lax.axis_index("c")
        c0 = wid * CS
        b = c0 // S
        pltpu.sync_copy(xyz_hbm.at[b], rq_v.at[pl.ds(0, 3)])
        pltpu.sync_copy(cent_hbm.at[pl.ds(c0, CS)], cent_v)

        def prep(i, carry):
            base = i * 16
            vx = rq_v[0, pl.ds(base, 16)]
            vy = rq_v[1, pl.ds(base, 16)]
            vz = rq_v[2, pl.ds(base, 16)]
            rq_v[0, pl.ds(base, 16)] = bf16r(vx)
            rq_v[1, pl.ds(base, 16)] = bf16r(vy)
            rq_v[2, pl.ds(base, 16)] = bf16r(vz)
            rq_v[3, pl.ds(base, 16)] = (vx * vx + vy * vy) + vz * vz
            return carry

        lax.fori_loop(0, N // 16, prep, jnp.int32(0))

        def centroid(j, carry):
            crow = cent_v[j, pl.ds(0, 16)]
            crr = bf16r(crow)
            cx = crow[0]
            cy = crow[1]
            cz = crow[2]
            cxr = crr[0]
            cyr = crr[1]
            czr = crr[2]
            szq = (cx * cx + cy * cy) + cz * cz

            r2max = max(r2s)

            def chunk(i, cnts):
                base = i * 16
                vx = rq_v[0, pl.ds(base, 16)]
                vy = rq_v[1, pl.ds(base, 16)]
                vz = rq_v[2, pl.ds(base, 16)]
                nq = rq_v[3, pl.ds(base, 16)]
                ip = (vx * cxr + vy * cyr) + vz * czr
                d = (-2.0 * ip + szq) + nq
                mmax = d <= r2max

                def hit(cnts):
                    iv = lax.iota(jnp.int32, 16) + base
                    new = []
                    for ri, (r2, K) in enumerate(zip(r2s, Ks)):
                        m = mmax if r2 == r2max else d <= r2
                        off = cnts[ri]
                        mi = m.astype(jnp.int32)
                        pos = plsc.cumsum(mi)
                        plsc.store_scatter(bufs[ri], [pos - 1 + off], iv,
                                           mask=m)
                        new.append(jnp.minimum(off + pos[15], K))
                    return tuple(new)

                return lax.cond(jnp.any(mmax), hit, lambda c: c, cnts)

            cnts = lax.fori_loop(0, N // 16, chunk,
                                 tuple(jnp.int32(0) for _ in Ks))
            for ri, K in enumerate(Ks):
                cnt = cnts[ri]
                first = jnp.where(cnt > 0, bufs[ri][pl.ds(0, 16)][0],
                                  jnp.int32(N - 1))
                for c in range(K // 16):
                    v = bufs[ri][pl.ds(c * 16, 16)]
                    pos = lax.iota(jnp.int32, 16) + c * 16
                    v = jnp.where(pos < cnt, v, first)
                    stage[ri][j, pl.ds(c * 16, 16)] = v
            return carry

        lax.fori_loop(0, CS, centroid, jnp.int32(0))
        for ri in range(len(Ks)):
            pltpu.sync_copy(stage[ri], outs[ri].at[pl.ds(c0, CS)])

    return k(xyz3, cent16)


def _stage1(h0, centrows, wt, bias, Df, interpret=False):
    R, D = h0.shape
    O = wt.shape[1]
    Rb = 2048
    assert R % Rb == 0

    def body(h_ref, c_ref, w_ref, b_ref, z_ref, st_ref):
        c3 = c_ref[:, 0:3]
        zpre = jnp.zeros((Rb, Df), jnp.float32)
        zpost = jnp.zeros((Rb, D - Df - 3), jnp.float32)
        sub = jnp.concatenate([zpre, c3, zpost], axis=1)
        hh = h_ref[...] - sub
        z = jnp.dot(hh, w_ref[...], preferred_element_type=jnp.float32)
        z = z + b_ref[0:1, :]
        z_ref[...] = z

        @pl.when(pl.program_id(0) == 0)
        def _():
            st_ref[...] = jnp.zeros_like(st_ref)

        s0 = jnp.sum(z, axis=0, keepdims=True)
        s1 = jnp.sum(z * z, axis=0, keepdims=True)
        pad = jnp.zeros((6, O), jnp.float32)
        st_ref[...] += jnp.concatenate([s0, s1, pad], axis=0)

    return pl.pallas_call(
        body,
        grid=(R // Rb,),
        in_specs=[
            pl.BlockSpec((Rb, D), lambda i: (i, 0)),
            pl.BlockSpec((Rb, 16), lambda i: (i, 0)),
            pl.BlockSpec((D, O), lambda i: (0, 0)),
            pl.BlockSpec((8, O), lambda i: (0, 0)),
        ],
        out_specs=[
            pl.BlockSpec((Rb, O), lambda i: (i, 0)),
            pl.BlockSpec((8, O), lambda i: (0, 0)),
        ],
        out_shape=[
            jax.ShapeDtypeStruct((R, O), jnp.float32),
            jax.ShapeDtypeStruct((8, O), jnp.float32),
        ],
        interpret=interpret,
    )(h0, centrows, wt, bias)


def _stage_mid(z, auxp, wt, bias, interpret=False):
    R, Op = z.shape
    On = wt.shape[1]
    Rb = 2048
    assert R % Rb == 0

    def body(z_ref, a_ref, w_ref, b_ref, z2_ref, st_ref):
        zz = z_ref[...]
        h = (zz - a_ref[0:1, :]) * a_ref[1:2, :] + a_ref[2:3, :]
        h = jnp.maximum(h, 0.0)
        z2 = jnp.dot(h, w_ref[...], preferred_element_type=jnp.float32)
        z2 = z2 + b_ref[0:1, :]
        z2_ref[...] = z2

        @pl.when(pl.program_id(0) == 0)
        def _():
            st_ref[...] = jnp.zeros_like(st_ref)

        s0 = jnp.sum(z2, axis=0, keepdims=True)
        s1 = jnp.sum(z2 * z2, axis=0, keepdims=True)
        pad = jnp.zeros((6, On), jnp.float32)
        st_ref[...] += jnp.concatenate([s0, s1, pad], axis=0)

    return pl.pallas_call(
        body,
        grid=(R // Rb,),
        in_specs=[
            pl.BlockSpec((Rb, Op), lambda i: (i, 0)),
            pl.BlockSpec((8, Op), lambda i: (0, 0)),
            pl.BlockSpec((Op, On), lambda i: (0, 0)),
            pl.BlockSpec((8, On), lambda i: (0, 0)),
        ],
        out_specs=[
            pl.BlockSpec((Rb, On), lambda i: (i, 0)),
            pl.BlockSpec((8, On), lambda i: (0, 0)),
        ],
        out_shape=[
            jax.ShapeDtypeStruct((R, On), jnp.float32),
            jax.ShapeDtypeStruct((8, On), jnp.float32),
        ],
        interpret=interpret,
    )(z, auxp, wt, bias)


def _stage_last(z3, auxp, K, interpret=False):
    BS_K, O = z3.shape
    BS = BS_K // K
    z3 = z3.reshape(BS, K, O)
    Sb = 1
    while Sb * 2 * K * O * 4 <= 2 * 1024 * 1024 and BS % (Sb * 2) == 0:
        Sb *= 2

    def body(z_ref, a_ref, o_ref):
        zz = z_ref[...]
        mean = a_ref[0:1, :][None]
        scale = a_ref[1:2, :][None]
        beta = a_ref[2:3, :][None]
        h = (zz - mean) * scale + beta
        h = jnp.maximum(h, 0.0)
        o_ref[...] = jnp.max(h, axis=1)

    return pl.pallas_call(
        body,
        grid=(BS // Sb,),
        in_specs=[
            pl.BlockSpec((Sb, K, O), lambda i: (i, 0, 0)),
            pl.BlockSpec((8, O), lambda i: (0, 0)),
        ],
        out_specs=pl.BlockSpec((Sb, O), lambda i: (i, 0)),
        out_shape=jax.ShapeDtypeStruct((BS, O), jnp.float32),
        interpret=interpret,
    )(z3, auxp)


def _bn_aux(stats, count, g, beta):
    mean = stats[0] / count
    var = stats[1] / count - mean * mean
    scale = g / jnp.sqrt(var + 1e-5)
    rows = jnp.stack([mean, scale, beta])
    return jnp.concatenate([rows, jnp.zeros((5, mean.shape[0]), jnp.float32)], 0)


def _bias_rows(b, O):
    return jnp.concatenate([b[None, :], jnp.zeros((7, O), jnp.float32)], axis=0)


def _sa_layer(cfg, params, xyz3, feats, interpret=False):
    B, _, N = xyz3.shape
    S = cfg['npoint']
    Df = feats.shape[1]
    D = _round_up(Df + 3, 16)

    fps_idx = _fps(xyz3, S, interpret=interpret)
    flat_fps = (fps_idx + jnp.arange(B, dtype=jnp.int32)[:, None] * N).reshape(-1)

    xyz_rows = jnp.transpose(xyz3, (0, 2, 1)).reshape(B * N, 3)
    table = jnp.concatenate(
        [feats, xyz_rows, jnp.zeros((B * N, D - Df - 3), jnp.float32)], axis=1)

    xyz16 = jnp.concatenate(
        [xyz_rows, jnp.zeros((B * N, 13), jnp.float32)], axis=1)
    cent16 = _sc_gather(xyz16, flat_fps)

    gis = _sc_ballquery(xyz3, cent16, S, cfg['radius_list'],
                        cfg['nsample_list'])

    outs = []
    for i, K in enumerate(cfg['nsample_list']):
        gi = gis[i]
        bbias = (jnp.arange(B * S, dtype=jnp.int32)[:, None] // S) * N
        flat_gi = (gi + bbias).reshape(-1)
        h0 = _sc_gather(table, flat_gi)
        centrows = jnp.broadcast_to(cent16[:, None, :],
                                    (B * S, K, 16)).reshape(B * S * K, 16)
        R = B * S * K

        branch = params[i]
        (w1, b1, g1, be1) = branch[0]
        O1 = w1.shape[0]
        w1t = jnp.zeros((D, O1), jnp.float32).at[:w1.shape[1]].set(w1.T)
        z1, st1 = _stage1(h0, centrows, w1t, _bias_rows(b1, O1), Df,
                          interpret=interpret)
        aux1 = _bn_aux(st1, R, g1, be1)

        (w2, b2, g2, be2) = branch[1]
        O2 = w2.shape[0]
        z2, st2 = _stage_mid(z1, aux1, w2.T, _bias_rows(b2, O2),
                             interpret=interpret)
        aux2 = _bn_aux(st2, R, g2, be2)

        (w3, b3, g3, be3) = branch[2]
        O3 = w3.shape[0]
        z3, st3 = _stage_mid(z2, aux2, w3.T, _bias_rows(b3, O3),
                             interpret=interpret)
        aux3 = _bn_aux(st3, R, g3, be3)

        outs.append(_stage_last(z3, aux3, K, interpret=interpret))

    new_feats = jnp.concatenate(outs, axis=1)
    new_xyz3 = jnp.transpose(cent16[:, :3].reshape(B, S, 3), (0, 2, 1))
    return new_xyz3, new_feats


def kernel(xyz, params1, params2, params3):
    B, C, N = xyz.shape
    xyz3 = xyz[:, :3, :]
    feats0 = jnp.transpose(xyz, (0, 2, 1)).reshape(B * N, C)
    x1, f1 = _sa_layer(_CFG[0], params1, xyz3, feats0)
    x2, f2 = _sa_layer(_CFG[1], params2, x1, f1)
    x3, f3 = _sa_layer(_CFG[2], params3, x2, f2)
    S3 = _CFG[2]['npoint']
    return jnp.transpose(f3.reshape(B, S3, -1), (0, 2, 1))


# speedup vs baseline: 1.1308x; 1.1308x over previous
import functools
import jax
import jax.numpy as jnp
import numpy as np
from jax import lax
from jax.experimental import pallas as pl
from jax.experimental.pallas import tpu as pltpu
from jax.experimental.pallas import tpu_sc as plsc

_ADD_CH = 3
_CFG = [
    dict(npoint=2048, radius_list=[0.05, 0.1, 0.2], nsample_list=[32, 64, 128],
         mlp_list=[[16, 16, 32], [32, 32, 64], [32, 48, 64]]),
    dict(npoint=512, radius_list=[0.2, 0.4], nsample_list=[64, 128],
         mlp_list=[[64, 64, 128], [64, 96, 128]]),
    dict(npoint=128, radius_list=[0.4, 0.8], nsample_list=[128, 256],
         mlp_list=[[128, 128, 256], [128, 196, 256]]),
]

_NC, _NS = 2, 16
_NW = _NC * _NS


def _round_up(x, m):
    return (x + m - 1) // m * m


def _fps_body(npoint, n_rows, n_cols, xs_ref, ys_ref, zs_ref, idx_ref, dist_ref):
    N = n_rows * n_cols
    rowi = lax.broadcasted_iota(jnp.int32, (n_rows, n_cols), 0)
    coli = lax.broadcasted_iota(jnp.int32, (n_rows, n_cols), 1)
    idxg = rowi * n_cols + coli
    dist_ref[...] = jnp.full((n_rows, n_cols), 1e10, dtype=jnp.float32)

    def step(t, far):
        idx_ref[t] = far
        sel = idxg == far
        cx = jnp.sum(jnp.where(sel, xs_ref[...], 0.0))
        cy = jnp.sum(jnp.where(sel, ys_ref[...], 0.0))
        cz = jnp.sum(jnp.where(sel, zs_ref[...], 0.0))
        dx = xs_ref[...] - cx
        dy = ys_ref[...] - cy
        dz = zs_ref[...] - cz
        d = dx * dx + dy * dy + dz * dz
        nd = jnp.minimum(dist_ref[...], d)
        dist_ref[...] = nd
        mx = jnp.max(nd)
        cand = jnp.where(nd == mx, idxg, N)
        return jnp.min(cand).astype(jnp.int32)

    lax.fori_loop(0, npoint, step, jnp.int32(0))


def _fps(xyz3, npoint, interpret=False):
    B, _, N = xyz3.shape
    n_rows = 8
    n_cols = N // 8
    xs = xyz3[:, 0, :].reshape(B, n_rows, n_cols)
    ys = xyz3[:, 1, :].reshape(B, n_rows, n_cols)
    zs = xyz3[:, 2, :].reshape(B, n_rows, n_cols)
    body = functools.partial(_fps_body, npoint, n_rows, n_cols)
    bs = pl.BlockSpec((1, n_rows, n_cols), lambda b: (b, 0, 0))
    out = pl.pallas_call(
        lambda x, y, z, i, dd: body(x.at[0], y.at[0], z.at[0], i.at[0, 0], dd),
        grid=(B,),
        in_specs=[bs, bs, bs],
        out_specs=pl.BlockSpec((1, 1, npoint), lambda b: (b, 0, 0),
                               memory_space=pltpu.SMEM),
        out_shape=jax.ShapeDtypeStruct((B, 1, npoint), jnp.int32),
        scratch_shapes=[pltpu.VMEM((n_rows, n_cols), jnp.float32)],
        interpret=interpret,
    )(xs, ys, zs)
    return out.reshape(B, npoint)


def _sc_gather(table, idx):
    T, D = table.shape
    F = idx.shape[0]
    assert F % (_NW * 8) == 0
    b_per_w = F // _NW
    chunk = min(128, b_per_w)
    nchunks = b_per_w // chunk
    assert b_per_w % chunk == 0
    depth = min(4, nchunks)
    while depth * chunk * D * 4 > 380 * 1024:
        depth //= 2
    ngroups = nchunks // depth
    assert nchunks % depth == 0
    idx2 = idx.reshape(F // chunk, chunk)
    mesh = plsc.VectorSubcoreMesh(core_axis_name="c", subcore_axis_name="s")

    @functools.partial(
        pl.kernel, mesh=mesh,
        out_type=jax.ShapeDtypeStruct((F, D), jnp.float32),
        scratch_types=[
            pltpu.VMEM((depth, chunk), jnp.int32),
            pltpu.VMEM((depth * chunk, D), jnp.float32),
            pltpu.SemaphoreType.DMA,
        ],
        compiler_params=pltpu.CompilerParams(use_tc_tiling_on_sc=False),
    )
    def k(table_hbm, idx_hbm, out_hbm, idx_v, rows_v, sem):
        wid = lax.axis_index("s") * _NC + lax.axis_index("c")
        gbase = wid * (b_per_w // chunk)

        def grp(g, carry):
            grow = gbase + g * depth
            pltpu.sync_copy(idx_hbm.at[pl.ds(grow, depth)], idx_v)
            cps = [
                pltpu.async_copy(table_hbm.at[idx_v.at[j]],
                                 rows_v.at[pl.ds(j * chunk, chunk)], sem)
                for j in range(depth)
            ]
            for cp in cps:
                cp.wait()
            pltpu.sync_copy(
                rows_v, out_hbm.at[pl.ds(grow * chunk, depth * chunk)])
            return carry

        lax.fori_loop(0, ngroups, grp, jnp.int32(0))

    return k(table, idx2)


def _sc_ballquery(xyz3, cent16, S, radii, Ks):
    B, _, N = xyz3.shape
    CS = (B * S) // _NW
    assert (B * S) % _NW == 0 and S % CS == 0
    r2s = [float(np.float32(r ** 2)) for r in radii]
    mesh = plsc.VectorSubcoreMesh(core_axis_name="c", subcore_axis_name="s")

    scratch = [pltpu.VMEM((4, N), jnp.float32),
               pltpu.VMEM((CS, 16), jnp.float32)]
    for K in Ks:
        scratch.append(pltpu.VMEM((K + 16,), jnp.int32))
    for K in Ks:
        scratch.append(pltpu.VMEM((CS, K), jnp.int32))

    @functools.partial(
        pl.kernel, mesh=mesh,
        out_type=[jax.ShapeDtypeStruct((B * S, K), jnp.int32) for K in Ks],
        scratch_types=scratch,
        compiler_params=pltpu.CompilerParams(needs_layout_passes=False),
    )
    def k(xyz_hbm, cent_hbm, *refs):
        outs = refs[:len(Ks)]
        rq_v = refs[len(Ks)]
        cent_v = refs[len(Ks) + 1]
        bufs = refs[len(Ks) + 2:len(Ks) + 2 + len(Ks)]
        stage = refs[len(Ks) + 2 + len(Ks):]

        def bf16r(x):
            xi = plsc.bitcast(x, jnp.int32)
            r = xi + (jnp.int32(0x7FFF) + ((xi >> 16) & 1))
            return plsc.bitcast(r & jnp.int32(-65536), jnp.float32)

        wid = lax.axis_index("s") * _NC + lax.axis_index("c")
        c0 = wid * CS
        b = c0 // S
        pltpu.sync_copy(xyz_hbm.at[b], rq_v.at[pl.ds(0, 3)])
        pltpu.sync_copy(cent_hbm.at[pl.ds(c0, CS)], cent_v)

        def prep(i, carry):
            base = i * 16
            vx = rq_v[0, pl.ds(base, 16)]
            vy = rq_v[1, pl.ds(base, 16)]
            vz = rq_v[2, pl.ds(base, 16)]
            rq_v[0, pl.ds(base, 16)] = bf16r(vx)
            rq_v[1, pl.ds(base, 16)] = bf16r(vy)
            rq_v[2, pl.ds(base, 16)] = bf16r(vz)
            rq_v[3, pl.ds(base, 16)] = (vx * vx + vy * vy) + vz * vz
            return carry

        lax.fori_loop(0, N // 16, prep, jnp.int32(0))

        def centroid(j, carry):
            crow = cent_v[j, pl.ds(0, 16)]
            crr = bf16r(crow)
            cx = crow[0]
            cy = crow[1]
            cz = crow[2]
            cxr = crr[0]
            cyr = crr[1]
            czr = crr[2]
            szq = (cx * cx + cy * cy) + cz * cz

            def chunk(i, cnts):
                base = i * 16
                vx = rq_v[0, pl.ds(base, 16)]
                vy = rq_v[1, pl.ds(base, 16)]
                vz = rq_v[2, pl.ds(base, 16)]
                nq = rq_v[3, pl.ds(base, 16)]
                ip = (vx * cxr + vy * cyr) + vz * czr
                d = (-2.0 * ip + szq) + nq
                iv = lax.iota(jnp.int32, 16) + base
                new = []
                for ri, (r2, K) in enumerate(zip(r2s, Ks)):
                    m = d <= r2
                    off = cnts[ri]
                    mi = m.astype(jnp.int32)
                    pos = plsc.cumsum(mi)
                    plsc.store_scatter(bufs[ri], [pos - 1 + off], iv, mask=m)
                    pc = plsc.all_reduce_population_count(m)
                    new.append(jnp.minimum(off + pc, K))
                return tuple(new)

            cnts = lax.fori_loop(0, N // 16, chunk,
                                 tuple(jnp.zeros((16,), jnp.int32)
                                       for _ in Ks))
            for ri, K in enumerate(Ks):
                cnt = cnts[ri][0]
                first = jnp.where(cnt > 0, bufs[ri][pl.ds(0, 16)][0],
                                  jnp.int32(N - 1))
                for c in range(K // 16):
                    v = bufs[ri][pl.ds(c * 16, 16)]
                    pos = lax.iota(jnp.int32, 16) + c * 16
                    v = jnp.where(pos < cnt, v, first)
                    stage[ri][j, pl.ds(c * 16, 16)] = v
            return carry

        lax.fori_loop(0, CS, centroid, jnp.int32(0))
        for ri in range(len(Ks)):
            pltpu.sync_copy(stage[ri], outs[ri].at[pl.ds(c0, CS)])

    return k(xyz3, cent16)


def _stage1(h0, centrows, wt, bias, Df, interpret=False):
    R, D = h0.shape
    O = wt.shape[1]
    Rb = 2048
    assert R % Rb == 0

    def body(h_ref, c_ref, w_ref, b_ref, z_ref, st_ref):
        c3 = c_ref[:, 0:3]
        zpre = jnp.zeros((Rb, Df), jnp.float32)
        zpost = jnp.zeros((Rb, D - Df - 3), jnp.float32)
        sub = jnp.concatenate([zpre, c3, zpost], axis=1)
        hh = h_ref[...] - sub
        z = jnp.dot(hh, w_ref[...], preferred_element_type=jnp.float32)
        z = z + b_ref[0:1, :]
        z_ref[...] = z

        @pl.when(pl.program_id(0) == 0)
        def _():
            st_ref[...] = jnp.zeros_like(st_ref)

        s0 = jnp.sum(z, axis=0, keepdims=True)
        s1 = jnp.sum(z * z, axis=0, keepdims=True)
        pad = jnp.zeros((6, O), jnp.float32)
        st_ref[...] += jnp.concatenate([s0, s1, pad], axis=0)

    return pl.pallas_call(
        body,
        grid=(R // Rb,),
        in_specs=[
            pl.BlockSpec((Rb, D), lambda i: (i, 0)),
            pl.BlockSpec((Rb, 16), lambda i: (i, 0)),
            pl.BlockSpec((D, O), lambda i: (0, 0)),
            pl.BlockSpec((8, O), lambda i: (0, 0)),
        ],
        out_specs=[
            pl.BlockSpec((Rb, O), lambda i: (i, 0)),
            pl.BlockSpec((8, O), lambda i: (0, 0)),
        ],
        out_shape=[
            jax.ShapeDtypeStruct((R, O), jnp.float32),
            jax.ShapeDtypeStruct((8, O), jnp.float32),
        ],
        interpret=interpret,
    )(h0, centrows, wt, bias)


def _stage_mid(z, auxp, wt, bias, interpret=False):
    R, Op = z.shape
    On = wt.shape[1]
    Rb = 2048
    assert R % Rb == 0

    def body(z_ref, a_ref, w_ref, b_ref, z2_ref, st_ref):
        zz = z_ref[...]
        h = (zz - a_ref[0:1, :]) * a_ref[1:2, :] + a_ref[2:3, :]
        h = jnp.maximum(h, 0.0)
        z2 = jnp.dot(h, w_ref[...], preferred_element_type=jnp.float32)
        z2 = z2 + b_ref[0:1, :]
        z2_ref[...] = z2

        @pl.when(pl.program_id(0) == 0)
        def _():
            st_ref[...] = jnp.zeros_like(st_ref)

        s0 = jnp.sum(z2, axis=0, keepdims=True)
        s1 = jnp.sum(z2 * z2, axis=0, keepdims=True)
        pad = jnp.zeros((6, On), jnp.float32)
        st_ref[...] += jnp.concatenate([s0, s1, pad], axis=0)

    return pl.pallas_call(
        body,
        grid=(R // Rb,),
        in_specs=[
            pl.BlockSpec((Rb, Op), lambda i: (i, 0)),
            pl.BlockSpec((8, Op), lambda i: (0, 0)),
            pl.BlockSpec((Op, On), lambda i: (0, 0)),
            pl.BlockSpec((8, On), lambda i: (0, 0)),
        ],
        out_specs=[
            pl.BlockSpec((Rb, On), lambda i: (i, 0)),
            pl.BlockSpec((8, On), lambda i: (0, 0)),
        ],
        out_shape=[
            jax.ShapeDtypeStruct((R, On), jnp.float32),
            jax.ShapeDtypeStruct((8, On), jnp.float32),
        ],
        interpret=interpret,
    )(z, auxp, wt, bias)


def _stage_last(z3, auxp, K, interpret=False):
    BS_K, O = z3.shape
    BS = BS_K // K
    z3 = z3.reshape(BS, K, O)
    Sb = 1
    while Sb * 2 * K * O * 4 <= 2 * 1024 * 1024 and BS % (Sb * 2) == 0:
        Sb *= 2

    def body(z_ref, a_ref, o_ref):
        zz = z_ref[...]
        mean = a_ref[0:1, :][None]
        scale = a_ref[1:2, :][None]
        beta = a_ref[2:3, :][None]
        h = (zz - mean) * scale + beta
        h = jnp.maximum(h, 0.0)
        o_ref[...] = jnp.max(h, axis=1)

    return pl.pallas_call(
        body,
        grid=(BS // Sb,),
        in_specs=[
            pl.BlockSpec((Sb, K, O), lambda i: (i, 0, 0)),
            pl.BlockSpec((8, O), lambda i: (0, 0)),
        ],
        out_specs=pl.BlockSpec((Sb, O), lambda i: (i, 0)),
        out_shape=jax.ShapeDtypeStruct((BS, O), jnp.float32),
        interpret=interpret,
    )(z3, auxp)


def _bn_aux(stats, count, g, beta):
    mean = stats[0] / count
    var = stats[1] / count - mean * mean
    scale = g / jnp.sqrt(var + 1e-5)
    rows = jnp.stack([mean, scale, beta])
    return jnp.concatenate([rows, jnp.zeros((5, mean.shape[0]), jnp.float32)], 0)


def _bias_rows(b, O):
    return jnp.concatenate([b[None, :], jnp.zeros((7, O), jnp.float32)], axis=0)


def _sa_layer(cfg, params, xyz3, feats, interpret=False):
    B, _, N = xyz3.shape
    S = cfg['npoint']
    Df = feats.shape[1]
    D = _round_up(Df + 3, 16)

    fps_idx = _fps(xyz3, S, interpret=interpret)
    flat_fps = (fps_idx + jnp.arange(B, dtype=jnp.int32)[:, None] * N).reshape(-1)

    xyz_rows = jnp.transpose(xyz3, (0, 2, 1)).reshape(B * N, 3)
    table = jnp.concatenate(
        [feats, xyz_rows, jnp.zeros((B * N, D - Df - 3), jnp.float32)], axis=1)

    xyz16 = jnp.concatenate(
        [xyz_rows, jnp.zeros((B * N, 13), jnp.float32)], axis=1)
    cent16 = _sc_gather(xyz16, flat_fps)

    gis = _sc_ballquery(xyz3, cent16, S, cfg['radius_list'],
                        cfg['nsample_list'])

    outs = []
    for i, K in enumerate(cfg['nsample_list']):
        gi = gis[i]
        bbias = (jnp.arange(B * S, dtype=jnp.int32)[:, None] // S) * N
        flat_gi = (gi + bbias).reshape(-1)
        h0 = _sc_gather(table, flat_gi)
        centrows = jnp.broadcast_to(cent16[:, None, :],
                                    (B * S, K, 16)).reshape(B * S * K, 16)
        R = B * S * K

        branch = params[i]
        (w1, b1, g1, be1) = branch[0]
        O1 = w1.shape[0]
        w1t = jnp.zeros((D, O1), jnp.float32).at[:w1.shape[1]].set(w1.T)
        z1, st1 = _stage1(h0, centrows, w1t, _bias_rows(b1, O1), Df,
                          interpret=interpret)
        aux1 = _bn_aux(st1, R, g1, be1)

        (w2, b2, g2, be2) = branch[1]
        O2 = w2.shape[0]
        z2, st2 = _stage_mid(z1, aux1, w2.T, _bias_rows(b2, O2),
                             interpret=interpret)
        aux2 = _bn_aux(st2, R, g2, be2)

        (w3, b3, g3, be3) = branch[2]
        O3 = w3.shape[0]
        z3, st3 = _stage_mid(z2, aux2, w3.T, _bias_rows(b3, O3),
                             interpret=interpret)
        aux3 = _bn_aux(st3, R, g3, be3)

        outs.append(_stage_last(z3, aux3, K, interpret=interpret))

    new_feats = jnp.concatenate(outs, axis=1)
    new_xyz3 = jnp.transpose(cent16[:, :3].reshape(B, S, 3), (0, 2, 1))
    return new_xyz3, new_feats


def kernel(xyz, params1, params2, params3):
    B, C, N = xyz.shape
    xyz3 = xyz[:, :3, :]
    feats0 = jnp.transpose(xyz, (0, 2, 1)).reshape(B * N, C)
    x1, f1 = _sa_layer(_CFG[0], params1, xyz3, feats0)
    x2, f2 = _sa_layer(_CFG[1], params2, x1, f1)
    x3, f3 = _sa_layer(_CFG[2], params3, x2, f2)
    S3 = _CFG[2]['npoint']
    return jnp.transpose(f3.reshape(B, S3, -1), (0, 2, 1))
